# baseline (speedup 1.0000x reference)
import functools

import jax
import jax.numpy as jnp
from jax import lax
from jax.experimental import pallas as pl
from jax.experimental.pallas import tpu as pltpu

_VMEM = 56 * 1024 * 1024


def _gemm_kernel(x_ref, w_ref, b_ref, o_ref, *, relu):
    acc = jnp.dot(x_ref[...], w_ref[...], preferred_element_type=jnp.float32)
    acc = acc + b_ref[...]
    if relu:
        acc = jnp.maximum(acc, 0.0)
    o_ref[...] = acc.astype(o_ref.dtype)


def _row_tile(m, cap=1024):
    if m <= cap:
        return m
    for d in range(cap - cap % 16, 15, -16):
        if m % d == 0:
            return d
    return m


def gemm_bias(x, w, b, relu=True):
    m, k = x.shape
    n = w.shape[1]
    tm = _row_tile(m)
    out = pl.pallas_call(
        functools.partial(_gemm_kernel, relu=relu),
        out_shape=jax.ShapeDtypeStruct((m, n), jnp.bfloat16),
        grid=(m // tm,),
        in_specs=[
            pl.BlockSpec((tm, k), lambda i: (i, 0)),
            pl.BlockSpec((k, n), lambda i: (0, 0)),
            pl.BlockSpec((1, n), lambda i: (0, 0)),
        ],
        out_specs=pl.BlockSpec((tm, n), lambda i: (i, 0)),
        compiler_params=pltpu.CompilerParams(
            dimension_semantics=("parallel",), vmem_limit_bytes=_VMEM),
    )(x, w, b)
    return out


def _pad_groups(x, pad):
    ng, h, gn, wd, cin = x.shape
    p2 = pad + 1
    xp = jnp.pad(x, ((0, 0), (p2, p2), (0, 0), (0, 0), (0, 0)))
    return xp.reshape(ng, (h + 2 * p2) * gn * wd, cin)


def _conv_kernel(x_ref, w_ref, b_ref, o_ref, *, kz, pad, wd, bb, cin, relu):
    m = o_ref.shape[0]
    col = lax.broadcasted_iota(jnp.int32, (m, 1), 0) % wd
    acc = jnp.broadcast_to(b_ref[...], o_ref.shape).astype(jnp.float32)
    for kh in range(kz):
        for kw in range(kz):
            dw = kw - pad
            xs = x_ref[pl.ds((kh + 1) * bb + dw, m), :]
            if dw != 0:
                ok = (col >= max(0, -dw)) & (col < wd - max(0, dw))
                xs = jnp.where(ok, xs, jnp.zeros_like(xs))
            wk = w_ref[pl.ds((kh * kz + kw) * cin, cin), :]
            acc = acc + jnp.dot(xs, wk, preferred_element_type=jnp.float32)
    if relu:
        acc = jnp.maximum(acc, 0.0)
    o_ref[...] = acc.astype(o_ref.dtype)


def conv_same(x, w, b, kz, pad, relu=True):
    ng, h, gn, wd, cin = x.shape
    cout = w.shape[1]
    bb = gn * wd
    hp = h + 2 * pad + 2
    xf = _pad_groups(x, pad)
    out = pl.pallas_call(
        functools.partial(_conv_kernel, kz=kz, pad=pad, wd=wd, bb=bb,
                          cin=cin, relu=relu),
        out_shape=jax.ShapeDtypeStruct((ng, h * bb, cout), jnp.bfloat16),
        grid=(ng,),
        in_specs=[
            pl.BlockSpec((None, hp * bb, cin), lambda g: (g, 0, 0)),
            pl.BlockSpec((kz * kz * cin, cout), lambda g: (0, 0)),
            pl.BlockSpec((1, cout), lambda g: (0, 0)),
        ],
        out_specs=pl.BlockSpec((None, h * bb, cout), lambda g: (g, 0, 0)),
        compiler_params=pltpu.CompilerParams(
            dimension_semantics=("parallel",), vmem_limit_bytes=_VMEM),
    )(xf, w, b)
    return out.reshape(ng, h, gn, wd, cout)


def _pool1x1_kernel(x_ref, w_ref, b_ref, o_ref, *, wd, bb, relu):
    m = o_ref.shape[0]
    col = lax.broadcasted_iota(jnp.int32, (m, 1), 0) % wd
    pooled = None
    for kh in range(3):
        for kw in range(3):
            dw = kw - 1
            xs = x_ref[pl.ds((kh + 1) * bb + dw, m), :]
            if dw != 0:
                ok = (col >= max(0, -dw)) & (col < wd - max(0, dw))
                xs = jnp.where(ok, xs, jnp.zeros_like(xs))
            pooled = xs if pooled is None else jnp.maximum(pooled, xs)
    acc = jnp.dot(pooled, w_ref[...], preferred_element_type=jnp.float32)
    acc = acc + b_ref[...]
    if relu:
        acc = jnp.maximum(acc, 0.0)
    o_ref[...] = acc.astype(o_ref.dtype)


def pool_conv1x1(x, w, b, relu=True):
    ng, h, gn, wd, cin = x.shape
    cout = w.shape[1]
    bb = gn * wd
    hp = h + 4
    xf = _pad_groups(x, 1)
    out = pl.pallas_call(
        functools.partial(_pool1x1_kernel, wd=wd, bb=bb, relu=relu),
        out_shape=jax.ShapeDtypeStruct((ng, h * bb, cout), jnp.bfloat16),
        grid=(ng,),
        in_specs=[
            pl.BlockSpec((None, hp * bb, cin), lambda g: (g, 0, 0)),
            pl.BlockSpec((cin, cout), lambda g: (0, 0)),
            pl.BlockSpec((1, cout), lambda g: (0, 0)),
        ],
        out_specs=pl.BlockSpec((None, h * bb, cout), lambda g: (g, 0, 0)),
        compiler_params=pltpu.CompilerParams(
            dimension_semantics=("parallel",), vmem_limit_bytes=_VMEM),
    )(xf, w, b)
    return out.reshape(ng, h, gn, wd, cout)


def max_pool5(x, k, s, p):
    return lax.reduce_window(
        x, jnp.asarray(-jnp.inf, x.dtype), lax.max,
        (1, k, 1, k, 1), (1, s, 1, s, 1),
        ((0, 0), (p, p), (0, 0), (p, p), (0, 0)))


def lrn_size2(x, alpha=1e-4, beta=0.75, k=1.0):
    xf = x.astype(jnp.float32)
    sq = xf * xf
    prev = jnp.pad(sq, [(0, 0)] * (x.ndim - 1) + [(1, 0)])[..., :-1]
    div = (sq + prev) * 0.5
    return (xf / jnp.power(k + alpha * div, beta)).astype(x.dtype)


def regroup(x, gn_new):
    ng, h, gn, wd, c = x.shape
    n = ng * gn
    x = x.transpose(0, 2, 1, 3, 4).reshape(n, h, wd, c)
    return x.reshape(n // gn_new, gn_new, h, wd, c).transpose(0, 2, 1, 3, 4)


_CFG = {
    "3a": (192, 64, 96, 128, 16, 32, 32),
    "3b": (256, 128, 128, 192, 32, 96, 64),
    "4a": (480, 192, 96, 208, 16, 48, 64),
    "4b": (512, 160, 112, 224, 24, 64, 64),
    "4c": (512, 128, 128, 256, 24, 64, 64),
    "4d": (512, 112, 144, 288, 32, 64, 64),
    "4e": (528, 256, 160, 320, 32, 128, 128),
    "5a": (832, 256, 160, 320, 32, 128, 128),
    "5b": (832, 384, 192, 384, 48, 128, 128),
}


def _inception(x, cfg, hw, hb, w2, b2, w3, b3, w4, b4):
    ch_in, c1, c3r, c5r = cfg[0], cfg[1], cfg[2], cfg[4]
    shp = x.shape
    head = gemm_bias(x.reshape(-1, ch_in), hw, hb, relu=True)
    head = head.reshape(*shp[:-1], c1 + c3r + c5r)
    y1 = head[..., :c1]
    y2 = conv_same(head[..., c1:c1 + c3r], w2, b2, 3, 1)
    y3 = conv_same(head[..., c1 + c3r:], w3, b3, 5, 2)
    y4 = pool_conv1x1(x, w4, b4)
    return jnp.concatenate([y1, y2, y3, y4], axis=-1)


def _stem_patches(x):
    cols = []
    for kh in range(7):
        for kw in range(7):
            cols.append(x[:, kh:kh + 223:2, kw:kw + 223:2, :])
    pat = jnp.concatenate(cols, axis=-1)
    return pat.reshape(x.shape[0] * 112 * 112, 7 * 7 * 3)


@jax.jit
def _forward(x_nchw, p):
    n = x_nchw.shape[0]
    x = jnp.transpose(x_nchw, (0, 2, 3, 1)).astype(jnp.bfloat16)
    xp = jnp.pad(x, ((0, 0), (3, 3), (3, 3), (0, 0)))
    y = gemm_bias(_stem_patches(xp), p["init1_w"], p["init1_b"], relu=True)
    x = y.reshape(n, 112, 1, 112, 64)
    x = max_pool5(x, 3, 2, 1)
    x = lrn_size2(x)
    y = gemm_bias(x.reshape(-1, 64), p["init2_w"], p["init2_b"])
    x = y.reshape(n, 56, 1, 56, 192)
    x = conv_same(x, p["init3_w"], p["init3_b"], 3, 1)
    x = lrn_size2(x)
    x = max_pool5(x, 3, 2, 1)
    x = regroup(x, 4)
    for name in ("3a", "3b"):
        x = _inception(x, _CFG[name], *p[name])
    x = max_pool5(x, 3, 2, 1)
    x = regroup(x, 16)
    for name in ("4a", "4b", "4c", "4d", "4e"):
        x = _inception(x, _CFG[name], *p[name])
    x = max_pool5(x, 3, 2, 1)
    for name in ("5a", "5b"):
        x = _inception(x, _CFG[name], *p[name])
    feat = x.astype(jnp.float32).sum(axis=(1, 3)) / 49.0
    feat = feat.reshape(n, 1024)
    logits = feat @ p["fc_w"].T + p["fc_b"]
    return jax.nn.softmax(logits, axis=1)


def kernel(x_nchw, init1_w, init1_b, init2_w, init2_b, init3_w, init3_b, in_3a_head_w, in_3a_head_b, in_3a_b2b_w, in_3a_b2b_b, in_3a_b3b_w, in_3a_b3b_b, in_3a_b4_w, in_3a_b4_b, in_3b_head_w, in_3b_head_b, in_3b_b2b_w, in_3b_b2b_b, in_3b_b3b_w, in_3b_b3b_b, in_3b_b4_w, in_3b_b4_b, in_4a_head_w, in_4a_head_b, in_4a_b2b_w, in_4a_b2b_b, in_4a_b3b_w, in_4a_b3b_b, in_4a_b4_w, in_4a_b4_b, in_4b_head_w, in_4b_head_b, in_4b_b2b_w, in_4b_b2b_b, in_4b_b3b_w, in_4b_b3b_b, in_4b_b4_w, in_4b_b4_b, in_4c_head_w, in_4c_head_b, in_4c_b2b_w, in_4c_b2b_b, in_4c_b3b_w, in_4c_b3b_b, in_4c_b4_w, in_4c_b4_b, in_4d_head_w, in_4d_head_b, in_4d_b2b_w, in_4d_b2b_b, in_4d_b3b_w, in_4d_b3b_b, in_4d_b4_w, in_4d_b4_b, in_4e_head_w, in_4e_head_b, in_4e_b2b_w, in_4e_b2b_b, in_4e_b3b_w, in_4e_b3b_b, in_4e_b4_w, in_4e_b4_b, in_5a_head_w, in_5a_head_b, in_5a_b2b_w, in_5a_b2b_b, in_5a_b3b_w, in_5a_b3b_b, in_5a_b4_w, in_5a_b4_b, in_5b_head_w, in_5b_head_b, in_5b_b2b_w, in_5b_b2b_b, in_5b_b3b_w, in_5b_b3b_b, in_5b_b4_w, in_5b_b4_b, fc_w, fc_b):
    p = {
        "init1_w": init1_w, "init1_b": init1_b,
        "init2_w": init2_w, "init2_b": init2_b,
        "init3_w": init3_w, "init3_b": init3_b,
        "3a": (in_3a_head_w, in_3a_head_b, in_3a_b2b_w, in_3a_b2b_b,
               in_3a_b3b_w, in_3a_b3b_b, in_3a_b4_w, in_3a_b4_b),
        "3b": (in_3b_head_w, in_3b_head_b, in_3b_b2b_w, in_3b_b2b_b,
               in_3b_b3b_w, in_3b_b3b_b, in_3b_b4_w, in_3b_b4_b),
        "4a": (in_4a_head_w, in_4a_head_b, in_4a_b2b_w, in_4a_b2b_b,
               in_4a_b3b_w, in_4a_b3b_b, in_4a_b4_w, in_4a_b4_b),
        "4b": (in_4b_head_w, in_4b_head_b, in_4b_b2b_w, in_4b_b2b_b,
               in_4b_b3b_w, in_4b_b3b_b, in_4b_b4_w, in_4b_b4_b),
        "4c": (in_4c_head_w, in_4c_head_b, in_4c_b2b_w, in_4c_b2b_b,
               in_4c_b3b_w, in_4c_b3b_b, in_4c_b4_w, in_4c_b4_b),
        "4d": (in_4d_head_w, in_4d_head_b, in_4d_b2b_w, in_4d_b2b_b,
               in_4d_b3b_w, in_4d_b3b_b, in_4d_b4_w, in_4d_b4_b),
        "4e": (in_4e_head_w, in_4e_head_b, in_4e_b2b_w, in_4e_b2b_b,
               in_4e_b3b_w, in_4e_b3b_b, in_4e_b4_w, in_4e_b4_b),
        "5a": (in_5a_head_w, in_5a_head_b, in_5a_b2b_w, in_5a_b2b_b,
               in_5a_b3b_w, in_5a_b3b_b, in_5a_b4_w, in_5a_b4_b),
        "5b": (in_5b_head_w, in_5b_head_b, in_5b_b2b_w, in_5b_b2b_b,
               in_5b_b3b_w, in_5b_b3b_b, in_5b_b4_w, in_5b_b4_b),
        "fc_w": fc_w, "fc_b": fc_b,
    }
    return _forward(x_nchw, p)


# speedup vs baseline: 2.3600x; 2.3600x over previous
import functools

import jax
import jax.numpy as jnp
from jax import lax
from jax.experimental import pallas as pl
from jax.experimental.pallas import tpu as pltpu

_VMEM = 56 * 1024 * 1024


def _gemm_kernel(x_ref, w_ref, b_ref, o_ref, *, relu):
    acc = jnp.dot(x_ref[...], w_ref[...], preferred_element_type=jnp.float32)
    acc = acc + b_ref[...]
    if relu:
        acc = jnp.maximum(acc, 0.0)
    o_ref[...] = acc.astype(o_ref.dtype)


def _row_tile(m, cap=1024):
    if m <= cap:
        return m
    for d in range(cap - cap % 16, 15, -16):
        if m % d == 0:
            return d
    return m


def gemm_bias(x, w, b, relu=True):
    m, k = x.shape
    n = w.shape[1]
    tm = _row_tile(m)
    out = pl.pallas_call(
        functools.partial(_gemm_kernel, relu=relu),
        out_shape=jax.ShapeDtypeStruct((m, n), jnp.bfloat16),
        grid=(m // tm,),
        in_specs=[
            pl.BlockSpec((tm, k), lambda i: (i, 0)),
            pl.BlockSpec((k, n), lambda i: (0, 0)),
            pl.BlockSpec((1, n), lambda i: (0, 0)),
        ],
        out_specs=pl.BlockSpec((tm, n), lambda i: (i, 0)),
        compiler_params=pltpu.CompilerParams(
            dimension_semantics=("parallel",), vmem_limit_bytes=_VMEM),
    )(x, w, b)
    return out


def _pad_groups(x, pad):
    ng, h, gn, wd, cin = x.shape
    p2 = pad + 1
    xp = jnp.pad(x, ((0, 0), (p2, p2), (0, 0), (0, 0), (0, 0)))
    return xp.reshape(ng, (h + 2 * p2) * gn * wd, cin)


def _conv_kernel(x_ref, w_ref, b_ref, o_ref, *, kz, pad, wd, bb, cin, relu):
    m = o_ref.shape[0]
    col = lax.broadcasted_iota(jnp.int32, (m, 1), 0) % wd
    acc = jnp.broadcast_to(b_ref[...], o_ref.shape).astype(jnp.float32)
    for kh in range(kz):
        for kw in range(kz):
            dw = kw - pad
            xs = x_ref[pl.ds((kh + 1) * bb + dw, m), :]
            if dw != 0:
                ok = (col >= max(0, -dw)) & (col < wd - max(0, dw))
                xs = jnp.where(ok, xs, jnp.zeros_like(xs))
            wk = w_ref[pl.ds((kh * kz + kw) * cin, cin), :]
            acc = acc + jnp.dot(xs, wk, preferred_element_type=jnp.float32)
    if relu:
        acc = jnp.maximum(acc, 0.0)
    o_ref[...] = acc.astype(o_ref.dtype)


def conv_same(x, w, b, kz, pad, relu=True):
    ng, h, gn, wd, cin = x.shape
    cout = w.shape[1]
    bb = gn * wd
    hp = h + 2 * pad + 2
    xf = _pad_groups(x, pad)
    out = pl.pallas_call(
        functools.partial(_conv_kernel, kz=kz, pad=pad, wd=wd, bb=bb,
                          cin=cin, relu=relu),
        out_shape=jax.ShapeDtypeStruct((ng, h * bb, cout), jnp.bfloat16),
        grid=(ng,),
        in_specs=[
            pl.BlockSpec((None, hp * bb, cin), lambda g: (g, 0, 0)),
            pl.BlockSpec((kz * kz * cin, cout), lambda g: (0, 0)),
            pl.BlockSpec((1, cout), lambda g: (0, 0)),
        ],
        out_specs=pl.BlockSpec((None, h * bb, cout), lambda g: (g, 0, 0)),
        compiler_params=pltpu.CompilerParams(
            dimension_semantics=("parallel",), vmem_limit_bytes=_VMEM),
    )(xf, w, b)
    return out.reshape(ng, h, gn, wd, cout)


def _pool1x1_kernel(x_ref, w_ref, b_ref, o_ref, *, wd, bb, relu):
    m = o_ref.shape[0]
    col = lax.broadcasted_iota(jnp.int32, (m, 1), 0) % wd
    pooled = None
    for kh in range(3):
        for kw in range(3):
            dw = kw - 1
            xs = x_ref[pl.ds((kh + 1) * bb + dw, m), :]
            if dw != 0:
                ok = (col >= max(0, -dw)) & (col < wd - max(0, dw))
                xs = jnp.where(ok, xs, jnp.zeros_like(xs))
            pooled = xs if pooled is None else jnp.maximum(pooled, xs)
    acc = jnp.dot(pooled, w_ref[...], preferred_element_type=jnp.float32)
    acc = acc + b_ref[...]
    if relu:
        acc = jnp.maximum(acc, 0.0)
    o_ref[...] = acc.astype(o_ref.dtype)


def pool_conv1x1(x, w, b, relu=True):
    ng, h, gn, wd, cin = x.shape
    cout = w.shape[1]
    bb = gn * wd
    hp = h + 4
    xf = _pad_groups(x, 1)
    out = pl.pallas_call(
        functools.partial(_pool1x1_kernel, wd=wd, bb=bb, relu=relu),
        out_shape=jax.ShapeDtypeStruct((ng, h * bb, cout), jnp.bfloat16),
        grid=(ng,),
        in_specs=[
            pl.BlockSpec((None, hp * bb, cin), lambda g: (g, 0, 0)),
            pl.BlockSpec((cin, cout), lambda g: (0, 0)),
            pl.BlockSpec((1, cout), lambda g: (0, 0)),
        ],
        out_specs=pl.BlockSpec((None, h * bb, cout), lambda g: (g, 0, 0)),
        compiler_params=pltpu.CompilerParams(
            dimension_semantics=("parallel",), vmem_limit_bytes=_VMEM),
    )(xf, w, b)
    return out.reshape(ng, h, gn, wd, cout)


def max_pool5(x, k, s, p):
    return lax.reduce_window(
        x, jnp.asarray(-jnp.inf, x.dtype), lax.max,
        (1, k, 1, k, 1), (1, s, 1, s, 1),
        ((0, 0), (p, p), (0, 0), (p, p), (0, 0)))


def lrn_size2(x, alpha=1e-4, beta=0.75, k=1.0):
    xf = x.astype(jnp.float32)
    sq = xf * xf
    prev = jnp.pad(sq, [(0, 0)] * (x.ndim - 1) + [(1, 0)])[..., :-1]
    div = (sq + prev) * 0.5
    return (xf / jnp.power(k + alpha * div, beta)).astype(x.dtype)


def regroup(x, gn_new):
    ng, h, gn, wd, c = x.shape
    n = ng * gn
    x = x.transpose(0, 2, 1, 3, 4).reshape(n, h, wd, c)
    return x.reshape(n // gn_new, gn_new, h, wd, c).transpose(0, 2, 1, 3, 4)


_CFG = {
    "3a": (192, 64, 96, 128, 16, 32, 32),
    "3b": (256, 128, 128, 192, 32, 96, 64),
    "4a": (480, 192, 96, 208, 16, 48, 64),
    "4b": (512, 160, 112, 224, 24, 64, 64),
    "4c": (512, 128, 128, 256, 24, 64, 64),
    "4d": (512, 112, 144, 288, 32, 64, 64),
    "4e": (528, 256, 160, 320, 32, 128, 128),
    "5a": (832, 256, 160, 320, 32, 128, 128),
    "5b": (832, 384, 192, 384, 48, 128, 128),
}


def _inception(x, cfg, hw, hb, w2, b2, w3, b3, w4, b4):
    ch_in, c1, c3r, c5r = cfg[0], cfg[1], cfg[2], cfg[4]
    shp = x.shape
    head = gemm_bias(x.reshape(-1, ch_in), hw, hb, relu=True)
    head = head.reshape(*shp[:-1], c1 + c3r + c5r)
    y1 = head[..., :c1]
    y2 = conv_same(head[..., c1:c1 + c3r], w2, b2, 3, 1)
    y3 = conv_same(head[..., c1 + c3r:], w3, b3, 5, 2)
    y4 = pool_conv1x1(x, w4, b4)
    return jnp.concatenate([y1, y2, y3, y4], axis=-1)


def _stem_gather_kernel(x_ref, o_ref):
    for kh in range(7):
        a, pp = divmod(kh + 1, 2)
        for kw in range(7):
            b, q = divmod(kw, 2)
            for ci in range(3):
                base = ((pp * 2 + q) * 3 + ci) * 116
                o_ref[(kh * 7 + kw) * 3 + ci, :, :] = (
                    x_ref[pl.ds(base + a, 112), pl.ds(b, 112)])


def _stem_mm_kernel(p_ref, w_ref, b_ref, o_ref):
    acc = lax.dot_general(p_ref[...], w_ref[...], (((0,), (0,)), ((), ())),
                          preferred_element_type=jnp.float32)
    acc = jnp.maximum(acc + b_ref[...], 0.0)
    o_ref[...] = acc.astype(o_ref.dtype)


def _stem(x_nchw, w, b):
    n = x_nchw.shape[0]
    xp = jnp.pad(x_nchw.astype(jnp.bfloat16),
                 ((0, 0), (0, 0), (4, 4), (3, 5)))
    planes = []
    for q in range(2):
        xq = xp[..., q::2]
        xq = xq.reshape(n, 3, 116, 2, 116)
        for pp in range(2):
            planes.append(xq[:, :, :, pp, :])
    xpl = jnp.stack([planes[q * 2 + pp][:, ci]
                     for pp in range(2) for q in range(2) for ci in range(3)],
                    axis=1)
    xpl = xpl.reshape(n, 12 * 116, 116)
    pt = pl.pallas_call(
        _stem_gather_kernel,
        out_shape=jax.ShapeDtypeStruct((n, 147, 112, 112), jnp.bfloat16),
        grid=(n,),
        in_specs=[pl.BlockSpec((None, 12 * 116, 116), lambda g: (g, 0, 0))],
        out_specs=pl.BlockSpec((None, 147, 112, 112), lambda g: (g, 0, 0, 0)),
        compiler_params=pltpu.CompilerParams(
            dimension_semantics=("parallel",), vmem_limit_bytes=_VMEM),
    )(xpl)
    pt = pt.reshape(n, 147, 112 * 112)
    out = pl.pallas_call(
        _stem_mm_kernel,
        out_shape=jax.ShapeDtypeStruct((n, 112 * 112, 64), jnp.bfloat16),
        grid=(n,),
        in_specs=[
            pl.BlockSpec((None, 147, 112 * 112), lambda g: (g, 0, 0)),
            pl.BlockSpec((147, 64), lambda g: (0, 0)),
            pl.BlockSpec((1, 64), lambda g: (0, 0)),
        ],
        out_specs=pl.BlockSpec((None, 112 * 112, 64), lambda g: (g, 0, 0)),
        compiler_params=pltpu.CompilerParams(
            dimension_semantics=("parallel",), vmem_limit_bytes=_VMEM),
    )(pt, w, b)
    return out.reshape(n, 112, 1, 112, 64)


@jax.jit
def _forward(x_nchw, p):
    n = x_nchw.shape[0]
    x = _stem(x_nchw, p["init1_w"], p["init1_b"])
    x = max_pool5(x, 3, 2, 1)
    x = lrn_size2(x)
    y = gemm_bias(x.reshape(-1, 64), p["init2_w"], p["init2_b"])
    x = y.reshape(n, 56, 1, 56, 192)
    x = conv_same(x, p["init3_w"], p["init3_b"], 3, 1)
    x = lrn_size2(x)
    x = max_pool5(x, 3, 2, 1)
    x = regroup(x, 4)
    for name in ("3a", "3b"):
        x = _inception(x, _CFG[name], *p[name])
    x = max_pool5(x, 3, 2, 1)
    x = regroup(x, 16)
    for name in ("4a", "4b", "4c", "4d", "4e"):
        x = _inception(x, _CFG[name], *p[name])
    x = max_pool5(x, 3, 2, 1)
    for name in ("5a", "5b"):
        x = _inception(x, _CFG[name], *p[name])
    feat = x.astype(jnp.float32).sum(axis=(1, 3)) / 49.0
    feat = feat.reshape(n, 1024)
    logits = feat @ p["fc_w"].T + p["fc_b"]
    return jax.nn.softmax(logits, axis=1)


def kernel(x_nchw, init1_w, init1_b, init2_w, init2_b, init3_w, init3_b, in_3a_head_w, in_3a_head_b, in_3a_b2b_w, in_3a_b2b_b, in_3a_b3b_w, in_3a_b3b_b, in_3a_b4_w, in_3a_b4_b, in_3b_head_w, in_3b_head_b, in_3b_b2b_w, in_3b_b2b_b, in_3b_b3b_w, in_3b_b3b_b, in_3b_b4_w, in_3b_b4_b, in_4a_head_w, in_4a_head_b, in_4a_b2b_w, in_4a_b2b_b, in_4a_b3b_w, in_4a_b3b_b, in_4a_b4_w, in_4a_b4_b, in_4b_head_w, in_4b_head_b, in_4b_b2b_w, in_4b_b2b_b, in_4b_b3b_w, in_4b_b3b_b, in_4b_b4_w, in_4b_b4_b, in_4c_head_w, in_4c_head_b, in_4c_b2b_w, in_4c_b2b_b, in_4c_b3b_w, in_4c_b3b_b, in_4c_b4_w, in_4c_b4_b, in_4d_head_w, in_4d_head_b, in_4d_b2b_w, in_4d_b2b_b, in_4d_b3b_w, in_4d_b3b_b, in_4d_b4_w, in_4d_b4_b, in_4e_head_w, in_4e_head_b, in_4e_b2b_w, in_4e_b2b_b, in_4e_b3b_w, in_4e_b3b_b, in_4e_b4_w, in_4e_b4_b, in_5a_head_w, in_5a_head_b, in_5a_b2b_w, in_5a_b2b_b, in_5a_b3b_w, in_5a_b3b_b, in_5a_b4_w, in_5a_b4_b, in_5b_head_w, in_5b_head_b, in_5b_b2b_w, in_5b_b2b_b, in_5b_b3b_w, in_5b_b3b_b, in_5b_b4_w, in_5b_b4_b, fc_w, fc_b):
    p = {
        "init1_w": init1_w, "init1_b": init1_b,
        "init2_w": init2_w, "init2_b": init2_b,
        "init3_w": init3_w, "init3_b": init3_b,
        "3a": (in_3a_head_w, in_3a_head_b, in_3a_b2b_w, in_3a_b2b_b,
               in_3a_b3b_w, in_3a_b3b_b, in_3a_b4_w, in_3a_b4_b),
        "3b": (in_3b_head_w, in_3b_head_b, in_3b_b2b_w, in_3b_b2b_b,
               in_3b_b3b_w, in_3b_b3b_b, in_3b_b4_w, in_3b_b4_b),
        "4a": (in_4a_head_w, in_4a_head_b, in_4a_b2b_w, in_4a_b2b_b,
               in_4a_b3b_w, in_4a_b3b_b, in_4a_b4_w, in_4a_b4_b),
        "4b": (in_4b_head_w, in_4b_head_b, in_4b_b2b_w, in_4b_b2b_b,
               in_4b_b3b_w, in_4b_b3b_b, in_4b_b4_w, in_4b_b4_b),
        "4c": (in_4c_head_w, in_4c_head_b, in_4c_b2b_w, in_4c_b2b_b,
               in_4c_b3b_w, in_4c_b3b_b, in_4c_b4_w, in_4c_b4_b),
        "4d": (in_4d_head_w, in_4d_head_b, in_4d_b2b_w, in_4d_b2b_b,
               in_4d_b3b_w, in_4d_b3b_b, in_4d_b4_w, in_4d_b4_b),
        "4e": (in_4e_head_w, in_4e_head_b, in_4e_b2b_w, in_4e_b2b_b,
               in_4e_b3b_w, in_4e_b3b_b, in_4e_b4_w, in_4e_b4_b),
        "5a": (in_5a_head_w, in_5a_head_b, in_5a_b2b_w, in_5a_b2b_b,
               in_5a_b3b_w, in_5a_b3b_b, in_5a_b4_w, in_5a_b4_b),
        "5b": (in_5b_head_w, in_5b_head_b, in_5b_b2b_w, in_5b_b2b_b,
               in_5b_b3b_w, in_5b_b3b_b, in_5b_b4_w, in_5b_b4_b),
        "fc_w": fc_w, "fc_b": fc_b,
    }
    return _forward(x_nchw, p)


# speedup vs baseline: 2.7622x; 1.1704x over previous
import functools

import jax
import jax.numpy as jnp
from jax import lax
from jax.experimental import pallas as pl
from jax.experimental.pallas import tpu as pltpu

_VMEM = 56 * 1024 * 1024


def _lrn2(xf):
    sq = xf * xf
    prev = jnp.pad(sq, ((0, 0), (1, 0)))[:, :-1]
    div = (sq + prev) * 0.5
    return xf / jnp.power(1.0 + 1e-4 * div, 0.75)


def _gemm_kernel(x_ref, w_ref, b_ref, o_ref, *, relu, lrn_in):
    x = x_ref[...]
    if lrn_in:
        x = _lrn2(x.astype(jnp.float32)).astype(jnp.bfloat16)
    acc = jnp.dot(x, w_ref[...], preferred_element_type=jnp.float32)
    acc = acc + b_ref[...]
    if relu:
        acc = jnp.maximum(acc, 0.0)
    o_ref[...] = acc.astype(o_ref.dtype)


def _row_tile(m, cap=1024):
    if m <= cap:
        return m
    for d in range(cap - cap % 16, 15, -16):
        if m % d == 0:
            return d
    return m


def gemm_bias(x, w, b, relu=True, lrn_in=False):
    m, k = x.shape
    n = w.shape[1]
    tm = _row_tile(m)
    out = pl.pallas_call(
        functools.partial(_gemm_kernel, relu=relu, lrn_in=lrn_in),
        out_shape=jax.ShapeDtypeStruct((m, n), jnp.bfloat16),
        grid=(m // tm,),
        in_specs=[
            pl.BlockSpec((tm, k), lambda i: (i, 0)),
            pl.BlockSpec((k, n), lambda i: (0, 0)),
            pl.BlockSpec((1, n), lambda i: (0, 0)),
        ],
        out_specs=pl.BlockSpec((tm, n), lambda i: (i, 0)),
        compiler_params=pltpu.CompilerParams(
            dimension_semantics=("parallel",), vmem_limit_bytes=_VMEM),
    )(x, w, b)
    return out


def _pad_groups(x, pad):
    ng, h, gn, wd, cin = x.shape
    p2 = pad + 1
    xp = jnp.pad(x, ((0, 0), (p2, p2), (0, 0), (0, 0), (0, 0)))
    return xp.reshape(ng, (h + 2 * p2) * gn * wd, cin)


def _conv_kernel(x_ref, w_ref, b_ref, o_ref, *, kz, pad, wd, bb, cin, relu,
                 lrn_out):
    m = o_ref.shape[0]
    col = lax.broadcasted_iota(jnp.int32, (m, 1), 0) % wd
    acc = jnp.broadcast_to(b_ref[...], o_ref.shape).astype(jnp.float32)
    for kh in range(kz):
        for kw in range(kz):
            dw = kw - pad
            xs = x_ref[pl.ds((kh + 1) * bb + dw, m), :]
            if dw != 0:
                ok = (col >= max(0, -dw)) & (col < wd - max(0, dw))
                xs = jnp.where(ok, xs, jnp.zeros_like(xs))
            wk = w_ref[pl.ds((kh * kz + kw) * cin, cin), :]
            acc = acc + jnp.dot(xs, wk, preferred_element_type=jnp.float32)
    if relu:
        acc = jnp.maximum(acc, 0.0)
    if lrn_out:
        acc = _lrn2(acc.astype(jnp.bfloat16).astype(jnp.float32))
    o_ref[...] = acc.astype(o_ref.dtype)


def conv_same(x, w, b, kz, pad, relu=True, lrn_out=False):
    ng, h, gn, wd, cin = x.shape
    cout = w.shape[1]
    bb = gn * wd
    hp = h + 2 * pad + 2
    xf = _pad_groups(x, pad)
    out = pl.pallas_call(
        functools.partial(_conv_kernel, kz=kz, pad=pad, wd=wd, bb=bb,
                          cin=cin, relu=relu, lrn_out=lrn_out),
        out_shape=jax.ShapeDtypeStruct((ng, h * bb, cout), jnp.bfloat16),
        grid=(ng,),
        in_specs=[
            pl.BlockSpec((None, hp * bb, cin), lambda g: (g, 0, 0)),
            pl.BlockSpec((kz * kz * cin, cout), lambda g: (0, 0)),
            pl.BlockSpec((1, cout), lambda g: (0, 0)),
        ],
        out_specs=pl.BlockSpec((None, h * bb, cout), lambda g: (g, 0, 0)),
        compiler_params=pltpu.CompilerParams(
            dimension_semantics=("parallel",), vmem_limit_bytes=_VMEM),
    )(xf, w, b)
    return out.reshape(ng, h, gn, wd, cout)


def _pool1x1_kernel(x_ref, w_ref, b_ref, o_ref, *, wd, bb, relu):
    m = o_ref.shape[0]
    col = lax.broadcasted_iota(jnp.int32, (m, 1), 0) % wd
    pooled = None
    for kh in range(3):
        for kw in range(3):
            dw = kw - 1
            xs = x_ref[pl.ds((kh + 1) * bb + dw, m), :]
            if dw != 0:
                ok = (col >= max(0, -dw)) & (col < wd - max(0, dw))
                xs = jnp.where(ok, xs, jnp.zeros_like(xs))
            pooled = xs if pooled is None else jnp.maximum(pooled, xs)
    acc = jnp.dot(pooled, w_ref[...], preferred_element_type=jnp.float32)
    acc = acc + b_ref[...]
    if relu:
        acc = jnp.maximum(acc, 0.0)
    o_ref[...] = acc.astype(o_ref.dtype)


def pool_conv1x1(x, w, b, relu=True):
    ng, h, gn, wd, cin = x.shape
    cout = w.shape[1]
    bb = gn * wd
    hp = h + 4
    xf = _pad_groups(x, 1)
    out = pl.pallas_call(
        functools.partial(_pool1x1_kernel, wd=wd, bb=bb, relu=relu),
        out_shape=jax.ShapeDtypeStruct((ng, h * bb, cout), jnp.bfloat16),
        grid=(ng,),
        in_specs=[
            pl.BlockSpec((None, hp * bb, cin), lambda g: (g, 0, 0)),
            pl.BlockSpec((cin, cout), lambda g: (0, 0)),
            pl.BlockSpec((1, cout), lambda g: (0, 0)),
        ],
        out_specs=pl.BlockSpec((None, h * bb, cout), lambda g: (g, 0, 0)),
        compiler_params=pltpu.CompilerParams(
            dimension_semantics=("parallel",), vmem_limit_bytes=_VMEM),
    )(xf, w, b)
    return out.reshape(ng, h, gn, wd, cout)


def max_pool5(x, k, s, p):
    return lax.reduce_window(
        x, jnp.asarray(-jnp.inf, x.dtype), lax.max,
        (1, k, 1, k, 1), (1, s, 1, s, 1),
        ((0, 0), (p, p), (0, 0), (p, p), (0, 0)))


def lrn_size2(x, alpha=1e-4, beta=0.75, k=1.0):
    xf = x.astype(jnp.float32)
    sq = xf * xf
    prev = jnp.pad(sq, [(0, 0)] * (x.ndim - 1) + [(1, 0)])[..., :-1]
    div = (sq + prev) * 0.5
    return (xf / jnp.power(k + alpha * div, beta)).astype(x.dtype)


def regroup(x, gn_new):
    ng, h, gn, wd, c = x.shape
    n = ng * gn
    x = x.transpose(0, 2, 1, 3, 4).reshape(n, h, wd, c)
    return x.reshape(n // gn_new, gn_new, h, wd, c).transpose(0, 2, 1, 3, 4)


_CFG = {
    "3a": (192, 64, 96, 128, 16, 32, 32),
    "3b": (256, 128, 128, 192, 32, 96, 64),
    "4a": (480, 192, 96, 208, 16, 48, 64),
    "4b": (512, 160, 112, 224, 24, 64, 64),
    "4c": (512, 128, 128, 256, 24, 64, 64),
    "4d": (512, 112, 144, 288, 32, 64, 64),
    "4e": (528, 256, 160, 320, 32, 128, 128),
    "5a": (832, 256, 160, 320, 32, 128, 128),
    "5b": (832, 384, 192, 384, 48, 128, 128),
}


def _inception(x, cfg, hw, hb, w2, b2, w3, b3, w4, b4):
    ch_in, c1, c3r, c5r = cfg[0], cfg[1], cfg[2], cfg[4]
    shp = x.shape
    head = gemm_bias(x.reshape(-1, ch_in), hw, hb, relu=True)
    head = head.reshape(*shp[:-1], c1 + c3r + c5r)
    y1 = head[..., :c1]
    y2 = conv_same(head[..., c1:c1 + c3r], w2, b2, 3, 1)
    y3 = conv_same(head[..., c1 + c3r:], w3, b3, 5, 2)
    y4 = pool_conv1x1(x, w4, b4)
    return jnp.concatenate([y1, y2, y3, y4], axis=-1)


def _stem_gather_kernel(x_ref, o_ref):
    for kh in range(7):
        a, pp = divmod(kh + 1, 2)
        for kw in range(7):
            b, q = divmod(kw, 2)
            for ci in range(3):
                base = ((pp * 2 + q) * 3 + ci) * 116
                o_ref[(kh * 7 + kw) * 3 + ci, :, :] = (
                    x_ref[pl.ds(base + a, 112), pl.ds(b, 112)])


def _stem_mm_kernel(p_ref, w_ref, b_ref, o_ref):
    pt = p_ref[...].reshape(147, 112 * 112)
    acc = lax.dot_general(pt, w_ref[...], (((0,), (0,)), ((), ())),
                          preferred_element_type=jnp.float32)
    acc = jnp.maximum(acc + b_ref[...], 0.0)
    o_ref[...] = acc.astype(o_ref.dtype)


def _stem(x_nchw, w, b):
    n = x_nchw.shape[0]
    xp = jnp.pad(x_nchw.astype(jnp.bfloat16),
                 ((0, 0), (0, 0), (4, 4), (3, 5)))
    planes = []
    for q in range(2):
        xq = xp[..., q::2]
        xq = xq.reshape(n, 3, 116, 2, 116)
        for pp in range(2):
            planes.append(xq[:, :, :, pp, :])
    xpl = jnp.stack([planes[q * 2 + pp][:, ci]
                     for pp in range(2) for q in range(2) for ci in range(3)],
                    axis=1)
    xpl = xpl.reshape(n, 12 * 116, 116)
    pt = pl.pallas_call(
        _stem_gather_kernel,
        out_shape=jax.ShapeDtypeStruct((n, 147, 112, 112), jnp.bfloat16),
        grid=(n,),
        in_specs=[pl.BlockSpec((None, 12 * 116, 116), lambda g: (g, 0, 0))],
        out_specs=pl.BlockSpec((None, 147, 112, 112), lambda g: (g, 0, 0, 0)),
        compiler_params=pltpu.CompilerParams(
            dimension_semantics=("parallel",), vmem_limit_bytes=_VMEM),
    )(xpl)
    out = pl.pallas_call(
        _stem_mm_kernel,
        out_shape=jax.ShapeDtypeStruct((n, 112 * 112, 64), jnp.bfloat16),
        grid=(n,),
        in_specs=[
            pl.BlockSpec((None, 147, 112, 112), lambda g: (g, 0, 0, 0)),
            pl.BlockSpec((147, 64), lambda g: (0, 0)),
            pl.BlockSpec((1, 64), lambda g: (0, 0)),
        ],
        out_specs=pl.BlockSpec((None, 112 * 112, 64), lambda g: (g, 0, 0)),
        compiler_params=pltpu.CompilerParams(
            dimension_semantics=("parallel",), vmem_limit_bytes=_VMEM),
    )(pt, w, b)
    return out.reshape(n, 112, 1, 112, 64)


@jax.jit
def _forward(x_nchw, p):
    n = x_nchw.shape[0]
    x = _stem(x_nchw, p["init1_w"], p["init1_b"])
    x = max_pool5(x, 3, 2, 1)
    y = gemm_bias(x.reshape(-1, 64), p["init2_w"], p["init2_b"], lrn_in=True)
    x = y.reshape(n, 56, 1, 56, 192)
    x = conv_same(x, p["init3_w"], p["init3_b"], 3, 1, lrn_out=True)
    x = max_pool5(x, 3, 2, 1)
    x = regroup(x, 4)
    for name in ("3a", "3b"):
        x = _inception(x, _CFG[name], *p[name])
    x = max_pool5(x, 3, 2, 1)
    x = regroup(x, 16)
    for name in ("4a", "4b", "4c", "4d", "4e"):
        x = _inception(x, _CFG[name], *p[name])
    x = max_pool5(x, 3, 2, 1)
    for name in ("5a", "5b"):
        x = _inception(x, _CFG[name], *p[name])
    feat = x.astype(jnp.float32).sum(axis=(1, 3)) / 49.0
    feat = feat.reshape(n, 1024)
    logits = feat @ p["fc_w"].T + p["fc_b"]
    return jax.nn.softmax(logits, axis=1)


def kernel(x_nchw, init1_w, init1_b, init2_w, init2_b, init3_w, init3_b, in_3a_head_w, in_3a_head_b, in_3a_b2b_w, in_3a_b2b_b, in_3a_b3b_w, in_3a_b3b_b, in_3a_b4_w, in_3a_b4_b, in_3b_head_w, in_3b_head_b, in_3b_b2b_w, in_3b_b2b_b, in_3b_b3b_w, in_3b_b3b_b, in_3b_b4_w, in_3b_b4_b, in_4a_head_w, in_4a_head_b, in_4a_b2b_w, in_4a_b2b_b, in_4a_b3b_w, in_4a_b3b_b, in_4a_b4_w, in_4a_b4_b, in_4b_head_w, in_4b_head_b, in_4b_b2b_w, in_4b_b2b_b, in_4b_b3b_w, in_4b_b3b_b, in_4b_b4_w, in_4b_b4_b, in_4c_head_w, in_4c_head_b, in_4c_b2b_w, in_4c_b2b_b, in_4c_b3b_w, in_4c_b3b_b, in_4c_b4_w, in_4c_b4_b, in_4d_head_w, in_4d_head_b, in_4d_b2b_w, in_4d_b2b_b, in_4d_b3b_w, in_4d_b3b_b, in_4d_b4_w, in_4d_b4_b, in_4e_head_w, in_4e_head_b, in_4e_b2b_w, in_4e_b2b_b, in_4e_b3b_w, in_4e_b3b_b, in_4e_b4_w, in_4e_b4_b, in_5a_head_w, in_5a_head_b, in_5a_b2b_w, in_5a_b2b_b, in_5a_b3b_w, in_5a_b3b_b, in_5a_b4_w, in_5a_b4_b, in_5b_head_w, in_5b_head_b, in_5b_b2b_w, in_5b_b2b_b, in_5b_b3b_w, in_5b_b3b_b, in_5b_b4_w, in_5b_b4_b, fc_w, fc_b):
    p = {
        "init1_w": init1_w, "init1_b": init1_b,
        "init2_w": init2_w, "init2_b": init2_b,
        "init3_w": init3_w, "init3_b": init3_b,
        "3a": (in_3a_head_w, in_3a_head_b, in_3a_b2b_w, in_3a_b2b_b,
               in_3a_b3b_w, in_3a_b3b_b, in_3a_b4_w, in_3a_b4_b),
        "3b": (in_3b_head_w, in_3b_head_b, in_3b_b2b_w, in_3b_b2b_b,
               in_3b_b3b_w, in_3b_b3b_b, in_3b_b4_w, in_3b_b4_b),
        "4a": (in_4a_head_w, in_4a_head_b, in_4a_b2b_w, in_4a_b2b_b,
               in_4a_b3b_w, in_4a_b3b_b, in_4a_b4_w, in_4a_b4_b),
        "4b": (in_4b_head_w, in_4b_head_b, in_4b_b2b_w, in_4b_b2b_b,
               in_4b_b3b_w, in_4b_b3b_b, in_4b_b4_w, in_4b_b4_b),
        "4c": (in_4c_head_w, in_4c_head_b, in_4c_b2b_w, in_4c_b2b_b,
               in_4c_b3b_w, in_4c_b3b_b, in_4c_b4_w, in_4c_b4_b),
        "4d": (in_4d_head_w, in_4d_head_b, in_4d_b2b_w, in_4d_b2b_b,
               in_4d_b3b_w, in_4d_b3b_b, in_4d_b4_w, in_4d_b4_b),
        "4e": (in_4e_head_w, in_4e_head_b, in_4e_b2b_w, in_4e_b2b_b,
               in_4e_b3b_w, in_4e_b3b_b, in_4e_b4_w, in_4e_b4_b),
        "5a": (in_5a_head_w, in_5a_head_b, in_5a_b2b_w, in_5a_b2b_b,
               in_5a_b3b_w, in_5a_b3b_b, in_5a_b4_w, in_5a_b4_b),
        "5b": (in_5b_head_w, in_5b_head_b, in_5b_b2b_w, in_5b_b2b_b,
               in_5b_b3b_w, in_5b_b3b_b, in_5b_b4_w, in_5b_b4_b),
        "fc_w": fc_w, "fc_b": fc_b,
    }
    return _forward(x_nchw, p)


# speedup vs baseline: 2.7833x; 1.0076x over previous
import functools

import jax
import jax.numpy as jnp
from jax import lax
from jax.experimental import pallas as pl
from jax.experimental.pallas import tpu as pltpu

_VMEM = 56 * 1024 * 1024


def _lrn2(xf):
    sq = xf * xf
    prev = jnp.pad(sq, ((0, 0), (1, 0)))[:, :-1]
    div = (sq + prev) * 0.5
    return xf / jnp.power(1.0 + 1e-4 * div, 0.75)


def _gemm_kernel(x_ref, w_ref, b_ref, o_ref, *, relu, lrn_in):
    x = x_ref[...]
    if lrn_in:
        x = _lrn2(x.astype(jnp.float32)).astype(jnp.bfloat16)
    acc = jnp.dot(x, w_ref[...], preferred_element_type=jnp.float32)
    acc = acc + b_ref[...]
    if relu:
        acc = jnp.maximum(acc, 0.0)
    o_ref[...] = acc.astype(o_ref.dtype)


def _row_tile(m, cap=1024):
    if m <= cap:
        return m
    for d in range(cap - cap % 16, 15, -16):
        if m % d == 0:
            return d
    return m


def gemm_bias(x, w, b, relu=True, lrn_in=False):
    m, k = x.shape
    n = w.shape[1]
    tm = _row_tile(m)
    out = pl.pallas_call(
        functools.partial(_gemm_kernel, relu=relu, lrn_in=lrn_in),
        out_shape=jax.ShapeDtypeStruct((m, n), jnp.bfloat16),
        grid=(m // tm,),
        in_specs=[
            pl.BlockSpec((tm, k), lambda i: (i, 0)),
            pl.BlockSpec((k, n), lambda i: (0, 0)),
            pl.BlockSpec((1, n), lambda i: (0, 0)),
        ],
        out_specs=pl.BlockSpec((tm, n), lambda i: (i, 0)),
        compiler_params=pltpu.CompilerParams(
            dimension_semantics=("parallel",), vmem_limit_bytes=_VMEM),
    )(x, w, b)
    return out


def _pad_groups(x, pad):
    ng, h, gn, wd, cin = x.shape
    p2 = pad + 1
    xp = jnp.pad(x, ((0, 0), (p2, p2), (0, 0), (0, 0), (0, 0)))
    return xp.reshape(ng, (h + 2 * p2) * gn * wd, cin)


def _conv_kernel(x_ref, w_ref, b_ref, o_ref, *, kz, pad, wd, bb, cin, relu,
                 lrn_out):
    m = o_ref.shape[0]
    col = lax.broadcasted_iota(jnp.int32, (m, 1), 0) % wd
    acc = jnp.broadcast_to(b_ref[...], o_ref.shape).astype(jnp.float32)
    for kh in range(kz):
        for kw in range(kz):
            dw = kw - pad
            xs = x_ref[pl.ds((kh + 1) * bb + dw, m), :]
            if dw != 0:
                ok = (col >= max(0, -dw)) & (col < wd - max(0, dw))
                xs = jnp.where(ok, xs, jnp.zeros_like(xs))
            wk = w_ref[pl.ds((kh * kz + kw) * cin, cin), :]
            acc = acc + jnp.dot(xs, wk, preferred_element_type=jnp.float32)
    if relu:
        acc = jnp.maximum(acc, 0.0)
    if lrn_out:
        acc = _lrn2(acc.astype(jnp.bfloat16).astype(jnp.float32))
    o_ref[...] = acc.astype(o_ref.dtype)


def conv_same(x, w, b, kz, pad, relu=True, lrn_out=False):
    ng, h, gn, wd, cin = x.shape
    cout = w.shape[1]
    bb = gn * wd
    hp = h + 2 * pad + 2
    xf = _pad_groups(x, pad)
    out = pl.pallas_call(
        functools.partial(_conv_kernel, kz=kz, pad=pad, wd=wd, bb=bb,
                          cin=cin, relu=relu, lrn_out=lrn_out),
        out_shape=jax.ShapeDtypeStruct((ng, h * bb, cout), jnp.bfloat16),
        grid=(ng,),
        in_specs=[
            pl.BlockSpec((None, hp * bb, cin), lambda g: (g, 0, 0)),
            pl.BlockSpec((kz * kz * cin, cout), lambda g: (0, 0)),
            pl.BlockSpec((1, cout), lambda g: (0, 0)),
        ],
        out_specs=pl.BlockSpec((None, h * bb, cout), lambda g: (g, 0, 0)),
        compiler_params=pltpu.CompilerParams(
            dimension_semantics=("parallel",), vmem_limit_bytes=_VMEM),
    )(xf, w, b)
    return out.reshape(ng, h, gn, wd, cout)


def _pool1x1_kernel(x_ref, w_ref, b_ref, o_ref, *, wd, bb, relu):
    m = o_ref.shape[0]
    col = lax.broadcasted_iota(jnp.int32, (m, 1), 0) % wd
    pooled = None
    for kh in range(3):
        for kw in range(3):
            dw = kw - 1
            xs = x_ref[pl.ds((kh + 1) * bb + dw, m), :]
            if dw != 0:
                ok = (col >= max(0, -dw)) & (col < wd - max(0, dw))
                xs = jnp.where(ok, xs, jnp.zeros_like(xs))
            pooled = xs if pooled is None else jnp.maximum(pooled, xs)
    acc = jnp.dot(pooled, w_ref[...], preferred_element_type=jnp.float32)
    acc = acc + b_ref[...]
    if relu:
        acc = jnp.maximum(acc, 0.0)
    o_ref[...] = acc.astype(o_ref.dtype)


def pool_conv1x1(x, w, b, relu=True):
    ng, h, gn, wd, cin = x.shape
    cout = w.shape[1]
    bb = gn * wd
    hp = h + 4
    xf = _pad_groups(x, 1)
    out = pl.pallas_call(
        functools.partial(_pool1x1_kernel, wd=wd, bb=bb, relu=relu),
        out_shape=jax.ShapeDtypeStruct((ng, h * bb, cout), jnp.bfloat16),
        grid=(ng,),
        in_specs=[
            pl.BlockSpec((None, hp * bb, cin), lambda g: (g, 0, 0)),
            pl.BlockSpec((cin, cout), lambda g: (0, 0)),
            pl.BlockSpec((1, cout), lambda g: (0, 0)),
        ],
        out_specs=pl.BlockSpec((None, h * bb, cout), lambda g: (g, 0, 0)),
        compiler_params=pltpu.CompilerParams(
            dimension_semantics=("parallel",), vmem_limit_bytes=_VMEM),
    )(xf, w, b)
    return out.reshape(ng, h, gn, wd, cout)


def max_pool5(x, k, s, p):
    return lax.reduce_window(
        x, jnp.asarray(-jnp.inf, x.dtype), lax.max,
        (1, k, 1, k, 1), (1, s, 1, s, 1),
        ((0, 0), (p, p), (0, 0), (p, p), (0, 0)))


def lrn_size2(x, alpha=1e-4, beta=0.75, k=1.0):
    xf = x.astype(jnp.float32)
    sq = xf * xf
    prev = jnp.pad(sq, [(0, 0)] * (x.ndim - 1) + [(1, 0)])[..., :-1]
    div = (sq + prev) * 0.5
    return (xf / jnp.power(k + alpha * div, beta)).astype(x.dtype)


def regroup(x, gn_new):
    ng, h, gn, wd, c = x.shape
    n = ng * gn
    x = x.transpose(0, 2, 1, 3, 4).reshape(n, h, wd, c)
    return x.reshape(n // gn_new, gn_new, h, wd, c).transpose(0, 2, 1, 3, 4)


_CFG = {
    "3a": (192, 64, 96, 128, 16, 32, 32),
    "3b": (256, 128, 128, 192, 32, 96, 64),
    "4a": (480, 192, 96, 208, 16, 48, 64),
    "4b": (512, 160, 112, 224, 24, 64, 64),
    "4c": (512, 128, 128, 256, 24, 64, 64),
    "4d": (512, 112, 144, 288, 32, 64, 64),
    "4e": (528, 256, 160, 320, 32, 128, 128),
    "5a": (832, 256, 160, 320, 32, 128, 128),
    "5b": (832, 384, 192, 384, 48, 128, 128),
}


def _inception(x, cfg, hw, hb, w2, b2, w3, b3, w4, b4):
    ch_in, c1, c3r, c5r = cfg[0], cfg[1], cfg[2], cfg[4]
    shp = x.shape
    head = gemm_bias(x.reshape(-1, ch_in), hw, hb, relu=True)
    head = head.reshape(*shp[:-1], c1 + c3r + c5r)
    y1 = head[..., :c1]
    y2 = conv_same(head[..., c1:c1 + c3r], w2, b2, 3, 1)
    y3 = conv_same(head[..., c1 + c3r:], w3, b3, 5, 2)
    y4 = pool_conv1x1(x, w4, b4)
    return jnp.concatenate([y1, y2, y3, y4], axis=-1)


def _stem_kernel(x_ref, w_ref, b_ref, o_ref):
    slabs = []
    for kh in range(7):
        a, pp = divmod(kh + 1, 2)
        for kw in range(7):
            b, q = divmod(kw, 2)
            for ci in range(3):
                base = ((pp * 2 + q) * 3 + ci) * 116
                slabs.append(x_ref[pl.ds(base + a, 112), pl.ds(b, 112)])
    pt = jnp.stack(slabs).reshape(147, 112 * 112)
    acc = lax.dot_general(pt, w_ref[...], (((0,), (0,)), ((), ())),
                          preferred_element_type=jnp.float32)
    acc = jnp.maximum(acc + b_ref[...], 0.0)
    o_ref[...] = acc.astype(o_ref.dtype)


def _stem(x_nchw, w, b):
    n = x_nchw.shape[0]
    xp = jnp.pad(x_nchw.astype(jnp.bfloat16),
                 ((0, 0), (0, 0), (4, 4), (3, 5)))
    planes = []
    for q in range(2):
        xq = xp[..., q::2]
        xq = xq.reshape(n, 3, 116, 2, 116)
        for pp in range(2):
            planes.append(xq[:, :, :, pp, :])
    xpl = jnp.stack([planes[q * 2 + pp][:, ci]
                     for pp in range(2) for q in range(2) for ci in range(3)],
                    axis=1)
    xpl = xpl.reshape(n, 12 * 116, 116)
    out = pl.pallas_call(
        _stem_kernel,
        out_shape=jax.ShapeDtypeStruct((n, 112 * 112, 64), jnp.bfloat16),
        grid=(n,),
        in_specs=[
            pl.BlockSpec((None, 12 * 116, 116), lambda g: (g, 0, 0)),
            pl.BlockSpec((147, 64), lambda g: (0, 0)),
            pl.BlockSpec((1, 64), lambda g: (0, 0)),
        ],
        out_specs=pl.BlockSpec((None, 112 * 112, 64), lambda g: (g, 0, 0)),
        compiler_params=pltpu.CompilerParams(
            dimension_semantics=("parallel",), vmem_limit_bytes=_VMEM),
    )(xpl, w, b)
    return out.reshape(n, 112, 1, 112, 64)


@jax.jit
def _forward(x_nchw, p):
    n = x_nchw.shape[0]
    x = _stem(x_nchw, p["init1_w"], p["init1_b"])
    x = max_pool5(x, 3, 2, 1)
    y = gemm_bias(x.reshape(-1, 64), p["init2_w"], p["init2_b"], lrn_in=True)
    x = y.reshape(n, 56, 1, 56, 192)
    x = conv_same(x, p["init3_w"], p["init3_b"], 3, 1, lrn_out=True)
    x = max_pool5(x, 3, 2, 1)
    x = regroup(x, 4)
    for name in ("3a", "3b"):
        x = _inception(x, _CFG[name], *p[name])
    x = max_pool5(x, 3, 2, 1)
    x = regroup(x, 16)
    for name in ("4a", "4b", "4c", "4d", "4e"):
        x = _inception(x, _CFG[name], *p[name])
    x = max_pool5(x, 3, 2, 1)
    for name in ("5a", "5b"):
        x = _inception(x, _CFG[name], *p[name])
    feat = x.astype(jnp.float32).sum(axis=(1, 3)) / 49.0
    feat = feat.reshape(n, 1024)
    logits = feat @ p["fc_w"].T + p["fc_b"]
    return jax.nn.softmax(logits, axis=1)


def kernel(x_nchw, init1_w, init1_b, init2_w, init2_b, init3_w, init3_b, in_3a_head_w, in_3a_head_b, in_3a_b2b_w, in_3a_b2b_b, in_3a_b3b_w, in_3a_b3b_b, in_3a_b4_w, in_3a_b4_b, in_3b_head_w, in_3b_head_b, in_3b_b2b_w, in_3b_b2b_b, in_3b_b3b_w, in_3b_b3b_b, in_3b_b4_w, in_3b_b4_b, in_4a_head_w, in_4a_head_b, in_4a_b2b_w, in_4a_b2b_b, in_4a_b3b_w, in_4a_b3b_b, in_4a_b4_w, in_4a_b4_b, in_4b_head_w, in_4b_head_b, in_4b_b2b_w, in_4b_b2b_b, in_4b_b3b_w, in_4b_b3b_b, in_4b_b4_w, in_4b_b4_b, in_4c_head_w, in_4c_head_b, in_4c_b2b_w, in_4c_b2b_b, in_4c_b3b_w, in_4c_b3b_b, in_4c_b4_w, in_4c_b4_b, in_4d_head_w, in_4d_head_b, in_4d_b2b_w, in_4d_b2b_b, in_4d_b3b_w, in_4d_b3b_b, in_4d_b4_w, in_4d_b4_b, in_4e_head_w, in_4e_head_b, in_4e_b2b_w, in_4e_b2b_b, in_4e_b3b_w, in_4e_b3b_b, in_4e_b4_w, in_4e_b4_b, in_5a_head_w, in_5a_head_b, in_5a_b2b_w, in_5a_b2b_b, in_5a_b3b_w, in_5a_b3b_b, in_5a_b4_w, in_5a_b4_b, in_5b_head_w, in_5b_head_b, in_5b_b2b_w, in_5b_b2b_b, in_5b_b3b_w, in_5b_b3b_b, in_5b_b4_w, in_5b_b4_b, fc_w, fc_b):
    p = {
        "init1_w": init1_w, "init1_b": init1_b,
        "init2_w": init2_w, "init2_b": init2_b,
        "init3_w": init3_w, "init3_b": init3_b,
        "3a": (in_3a_head_w, in_3a_head_b, in_3a_b2b_w, in_3a_b2b_b,
               in_3a_b3b_w, in_3a_b3b_b, in_3a_b4_w, in_3a_b4_b),
        "3b": (in_3b_head_w, in_3b_head_b, in_3b_b2b_w, in_3b_b2b_b,
               in_3b_b3b_w, in_3b_b3b_b, in_3b_b4_w, in_3b_b4_b),
        "4a": (in_4a_head_w, in_4a_head_b, in_4a_b2b_w, in_4a_b2b_b,
               in_4a_b3b_w, in_4a_b3b_b, in_4a_b4_w, in_4a_b4_b),
        "4b": (in_4b_head_w, in_4b_head_b, in_4b_b2b_w, in_4b_b2b_b,
               in_4b_b3b_w, in_4b_b3b_b, in_4b_b4_w, in_4b_b4_b),
        "4c": (in_4c_head_w, in_4c_head_b, in_4c_b2b_w, in_4c_b2b_b,
               in_4c_b3b_w, in_4c_b3b_b, in_4c_b4_w, in_4c_b4_b),
        "4d": (in_4d_head_w, in_4d_head_b, in_4d_b2b_w, in_4d_b2b_b,
               in_4d_b3b_w, in_4d_b3b_b, in_4d_b4_w, in_4d_b4_b),
        "4e": (in_4e_head_w, in_4e_head_b, in_4e_b2b_w, in_4e_b2b_b,
               in_4e_b3b_w, in_4e_b3b_b, in_4e_b4_w, in_4e_b4_b),
        "5a": (in_5a_head_w, in_5a_head_b, in_5a_b2b_w, in_5a_b2b_b,
               in_5a_b3b_w, in_5a_b3b_b, in_5a_b4_w, in_5a_b4_b),
        "5b": (in_5b_head_w, in_5b_head_b, in_5b_b2b_w, in_5b_b2b_b,
               in_5b_b3b_w, in_5b_b3b_b, in_5b_b4_w, in_5b_b4_b),
        "fc_w": fc_w, "fc_b": fc_b,
    }
    return _forward(x_nchw, p)


# speedup vs baseline: 3.0614x; 1.0999x over previous
import functools

import jax
import jax.numpy as jnp
from jax import lax
from jax.experimental import pallas as pl
from jax.experimental.pallas import tpu as pltpu

_VMEM = 56 * 1024 * 1024


def _lrn2(xf):
    sq = xf * xf
    prev = jnp.pad(sq, ((0, 0), (1, 0)))[:, :-1]
    div = (sq + prev) * 0.5
    return xf / jnp.power(1.0 + 1e-4 * div, 0.75)


def _gemm_kernel(x_ref, w_ref, b_ref, o_ref, *, relu, lrn_in):
    x = x_ref[...]
    if lrn_in:
        x = _lrn2(x.astype(jnp.float32)).astype(jnp.bfloat16)
    acc = jnp.dot(x, w_ref[...], preferred_element_type=jnp.float32)
    acc = acc + b_ref[...]
    if relu:
        acc = jnp.maximum(acc, 0.0)
    o_ref[...] = acc.astype(o_ref.dtype)


def _row_tile(m, cap=1024):
    if m <= cap:
        return m
    for d in range(cap - cap % 16, 15, -16):
        if m % d == 0:
            return d
    return m


def gemm_bias(x, w, b, relu=True, lrn_in=False):
    m, k = x.shape
    n = w.shape[1]
    tm = _row_tile(m)
    out = pl.pallas_call(
        functools.partial(_gemm_kernel, relu=relu, lrn_in=lrn_in),
        out_shape=jax.ShapeDtypeStruct((m, n), jnp.bfloat16),
        grid=(m // tm,),
        in_specs=[
            pl.BlockSpec((tm, k), lambda i: (i, 0)),
            pl.BlockSpec((k, n), lambda i: (0, 0)),
            pl.BlockSpec((1, n), lambda i: (0, 0)),
        ],
        out_specs=pl.BlockSpec((tm, n), lambda i: (i, 0)),
        compiler_params=pltpu.CompilerParams(
            dimension_semantics=("parallel",), vmem_limit_bytes=_VMEM),
    )(x, w, b)
    return out


def _pad_groups(x, pad):
    ng, h, gn, wd, cin = x.shape
    p2 = pad + 1
    xp = jnp.pad(x, ((0, 0), (p2, p2), (0, 0), (0, 0), (0, 0)))
    return xp.reshape(ng, (h + 2 * p2) * gn * wd, cin)


def _conv_kernel(x_ref, w_ref, b_ref, o_ref, *, kz, pad, wd, bb, cin, relu,
                 lrn_out):
    m = o_ref.shape[0]
    col = lax.broadcasted_iota(jnp.int32, (m, 1), 0) % wd
    acc = jnp.broadcast_to(b_ref[...], o_ref.shape).astype(jnp.float32)
    for kh in range(kz):
        for kw in range(kz):
            dw = kw - pad
            xs = x_ref[pl.ds((kh + 1) * bb + dw, m), :]
            if dw != 0:
                ok = (col >= max(0, -dw)) & (col < wd - max(0, dw))
                xs = jnp.where(ok, xs, jnp.zeros_like(xs))
            wk = w_ref[pl.ds((kh * kz + kw) * cin, cin), :]
            acc = acc + jnp.dot(xs, wk, preferred_element_type=jnp.float32)
    if relu:
        acc = jnp.maximum(acc, 0.0)
    if lrn_out:
        acc = _lrn2(acc.astype(jnp.bfloat16).astype(jnp.float32))
    o_ref[...] = acc.astype(o_ref.dtype)


def conv_same(x, w, b, kz, pad, relu=True, lrn_out=False):
    ng, h, gn, wd, cin = x.shape
    cout = w.shape[1]
    bb = gn * wd
    hp = h + 2 * pad + 2
    xf = _pad_groups(x, pad)
    out = pl.pallas_call(
        functools.partial(_conv_kernel, kz=kz, pad=pad, wd=wd, bb=bb,
                          cin=cin, relu=relu, lrn_out=lrn_out),
        out_shape=jax.ShapeDtypeStruct((ng, h * bb, cout), jnp.bfloat16),
        grid=(ng,),
        in_specs=[
            pl.BlockSpec((None, hp * bb, cin), lambda g: (g, 0, 0)),
            pl.BlockSpec((kz * kz * cin, cout), lambda g: (0, 0)),
            pl.BlockSpec((1, cout), lambda g: (0, 0)),
        ],
        out_specs=pl.BlockSpec((None, h * bb, cout), lambda g: (g, 0, 0)),
        compiler_params=pltpu.CompilerParams(
            dimension_semantics=("parallel",), vmem_limit_bytes=_VMEM),
    )(xf, w, b)
    return out.reshape(ng, h, gn, wd, cout)


def _pool1x1_kernel(x_ref, w_ref, b_ref, o_ref, *, wd, bb, relu):
    m = o_ref.shape[0]
    col = lax.broadcasted_iota(jnp.int32, (m, 1), 0) % wd
    pooled = None
    for kh in range(3):
        for kw in range(3):
            dw = kw - 1
            xs = x_ref[pl.ds((kh + 1) * bb + dw, m), :]
            if dw != 0:
                ok = (col >= max(0, -dw)) & (col < wd - max(0, dw))
                xs = jnp.where(ok, xs, jnp.zeros_like(xs))
            pooled = xs if pooled is None else jnp.maximum(pooled, xs)
    acc = jnp.dot(pooled, w_ref[...], preferred_element_type=jnp.float32)
    acc = acc + b_ref[...]
    if relu:
        acc = jnp.maximum(acc, 0.0)
    o_ref[...] = acc.astype(o_ref.dtype)


def _headpool_kernel(x_ref, hw_ref, hb_ref, pw_ref, pb_ref, oh_ref, op_ref,
                     *, wd, bb):
    m = oh_ref.shape[0]
    col = lax.broadcasted_iota(jnp.int32, (m, 1), 0) % wd
    xc = x_ref[pl.ds(2 * bb, m), :]
    hacc = jnp.dot(xc, hw_ref[...], preferred_element_type=jnp.float32)
    oh_ref[...] = jnp.maximum(hacc + hb_ref[...], 0.0).astype(oh_ref.dtype)
    pooled = None
    for kh in range(3):
        for kw in range(3):
            dw = kw - 1
            xs = x_ref[pl.ds((kh + 1) * bb + dw, m), :]
            if dw != 0:
                ok = (col >= max(0, -dw)) & (col < wd - max(0, dw))
                xs = jnp.where(ok, xs, jnp.zeros_like(xs))
            pooled = xs if pooled is None else jnp.maximum(pooled, xs)
    pacc = jnp.dot(pooled, pw_ref[...], preferred_element_type=jnp.float32)
    op_ref[...] = jnp.maximum(pacc + pb_ref[...], 0.0).astype(op_ref.dtype)


def head_pool(x, hw, hb, pw, pb):
    ng, h, gn, wd, cin = x.shape
    nh, np_ = hw.shape[1], pw.shape[1]
    bb = gn * wd
    hp = h + 4
    xf = _pad_groups(x, 1)
    head, pooled = pl.pallas_call(
        functools.partial(_headpool_kernel, wd=wd, bb=bb),
        out_shape=[jax.ShapeDtypeStruct((ng, h * bb, nh), jnp.bfloat16),
                   jax.ShapeDtypeStruct((ng, h * bb, np_), jnp.bfloat16)],
        grid=(ng,),
        in_specs=[
            pl.BlockSpec((None, hp * bb, cin), lambda g: (g, 0, 0)),
            pl.BlockSpec((cin, nh), lambda g: (0, 0)),
            pl.BlockSpec((1, nh), lambda g: (0, 0)),
            pl.BlockSpec((cin, np_), lambda g: (0, 0)),
            pl.BlockSpec((1, np_), lambda g: (0, 0)),
        ],
        out_specs=[pl.BlockSpec((None, h * bb, nh), lambda g: (g, 0, 0)),
                   pl.BlockSpec((None, h * bb, np_), lambda g: (g, 0, 0))],
        compiler_params=pltpu.CompilerParams(
            dimension_semantics=("parallel",), vmem_limit_bytes=_VMEM),
    )(xf, hw, hb, pw, pb)
    return (head.reshape(ng, h, gn, wd, nh), pooled.reshape(ng, h, gn, wd, np_))


def pool_conv1x1(x, w, b, relu=True):
    ng, h, gn, wd, cin = x.shape
    cout = w.shape[1]
    bb = gn * wd
    hp = h + 4
    xf = _pad_groups(x, 1)
    out = pl.pallas_call(
        functools.partial(_pool1x1_kernel, wd=wd, bb=bb, relu=relu),
        out_shape=jax.ShapeDtypeStruct((ng, h * bb, cout), jnp.bfloat16),
        grid=(ng,),
        in_specs=[
            pl.BlockSpec((None, hp * bb, cin), lambda g: (g, 0, 0)),
            pl.BlockSpec((cin, cout), lambda g: (0, 0)),
            pl.BlockSpec((1, cout), lambda g: (0, 0)),
        ],
        out_specs=pl.BlockSpec((None, h * bb, cout), lambda g: (g, 0, 0)),
        compiler_params=pltpu.CompilerParams(
            dimension_semantics=("parallel",), vmem_limit_bytes=_VMEM),
    )(xf, w, b)
    return out.reshape(ng, h, gn, wd, cout)


def max_pool5(x, k, s, p):
    return lax.reduce_window(
        x, jnp.asarray(-jnp.inf, x.dtype), lax.max,
        (1, k, 1, k, 1), (1, s, 1, s, 1),
        ((0, 0), (p, p), (0, 0), (p, p), (0, 0)))


def lrn_size2(x, alpha=1e-4, beta=0.75, k=1.0):
    xf = x.astype(jnp.float32)
    sq = xf * xf
    prev = jnp.pad(sq, [(0, 0)] * (x.ndim - 1) + [(1, 0)])[..., :-1]
    div = (sq + prev) * 0.5
    return (xf / jnp.power(k + alpha * div, beta)).astype(x.dtype)


def regroup(x, gn_new):
    ng, h, gn, wd, c = x.shape
    n = ng * gn
    x = x.transpose(0, 2, 1, 3, 4).reshape(n, h, wd, c)
    return x.reshape(n // gn_new, gn_new, h, wd, c).transpose(0, 2, 1, 3, 4)


_CFG = {
    "3a": (192, 64, 96, 128, 16, 32, 32),
    "3b": (256, 128, 128, 192, 32, 96, 64),
    "4a": (480, 192, 96, 208, 16, 48, 64),
    "4b": (512, 160, 112, 224, 24, 64, 64),
    "4c": (512, 128, 128, 256, 24, 64, 64),
    "4d": (512, 112, 144, 288, 32, 64, 64),
    "4e": (528, 256, 160, 320, 32, 128, 128),
    "5a": (832, 256, 160, 320, 32, 128, 128),
    "5b": (832, 384, 192, 384, 48, 128, 128),
}


def _inception(x, cfg, hw, hb, w2, b2, w3, b3, w4, b4):
    c1, c3r = cfg[1], cfg[2]
    head, y4 = head_pool(x, hw, hb, w4, b4)
    y1 = head[..., :c1]
    y2 = conv_same(head[..., c1:c1 + c3r], w2, b2, 3, 1)
    y3 = conv_same(head[..., c1 + c3r:], w3, b3, 5, 2)
    return jnp.concatenate([y1, y2, y3, y4], axis=-1)


def _stem_kernel(x_ref, w_ref, b_ref, o_ref):
    slabs = []
    for kh in range(7):
        a, pp = divmod(kh + 1, 2)
        for kw in range(7):
            b, q = divmod(kw, 2)
            for ci in range(3):
                base = ((pp * 2 + q) * 3 + ci) * 116
                slabs.append(x_ref[pl.ds(base + a, 112), pl.ds(b, 112)])
    pt = jnp.stack(slabs).reshape(147, 112 * 112)
    acc = lax.dot_general(pt, w_ref[...], (((0,), (0,)), ((), ())),
                          preferred_element_type=jnp.float32)
    acc = jnp.maximum(acc + b_ref[...], 0.0)
    o_ref[...] = acc.astype(o_ref.dtype)


def _stem(x_nchw, w, b):
    n = x_nchw.shape[0]
    xp = jnp.pad(x_nchw.astype(jnp.bfloat16),
                 ((0, 0), (0, 0), (4, 4), (3, 5)))
    planes = []
    for q in range(2):
        xq = xp[..., q::2]
        xq = xq.reshape(n, 3, 116, 2, 116)
        for pp in range(2):
            planes.append(xq[:, :, :, pp, :])
    xpl = jnp.stack([planes[q * 2 + pp][:, ci]
                     for pp in range(2) for q in range(2) for ci in range(3)],
                    axis=1)
    xpl = xpl.reshape(n, 12 * 116, 116)
    out = pl.pallas_call(
        _stem_kernel,
        out_shape=jax.ShapeDtypeStruct((n, 112 * 112, 64), jnp.bfloat16),
        grid=(n,),
        in_specs=[
            pl.BlockSpec((None, 12 * 116, 116), lambda g: (g, 0, 0)),
            pl.BlockSpec((147, 64), lambda g: (0, 0)),
            pl.BlockSpec((1, 64), lambda g: (0, 0)),
        ],
        out_specs=pl.BlockSpec((None, 112 * 112, 64), lambda g: (g, 0, 0)),
        compiler_params=pltpu.CompilerParams(
            dimension_semantics=("parallel",), vmem_limit_bytes=_VMEM),
    )(xpl, w, b)
    return out.reshape(n, 112, 1, 112, 64)


@jax.jit
def _forward(x_nchw, p):
    n = x_nchw.shape[0]
    x = _stem(x_nchw, p["init1_w"], p["init1_b"])
    x = max_pool5(x, 3, 2, 1)
    y = gemm_bias(x.reshape(-1, 64), p["init2_w"], p["init2_b"], lrn_in=True)
    x = y.reshape(n, 56, 1, 56, 192)
    x = conv_same(x, p["init3_w"], p["init3_b"], 3, 1, lrn_out=True)
    x = max_pool5(x, 3, 2, 1)
    x = regroup(x, 4)
    for name in ("3a", "3b"):
        x = _inception(x, _CFG[name], *p[name])
    x = max_pool5(x, 3, 2, 1)
    x = regroup(x, 16)
    for name in ("4a", "4b", "4c", "4d", "4e"):
        x = _inception(x, _CFG[name], *p[name])
    x = max_pool5(x, 3, 2, 1)
    for name in ("5a", "5b"):
        x = _inception(x, _CFG[name], *p[name])
    feat = x.astype(jnp.float32).sum(axis=(1, 3)) / 49.0
    feat = feat.reshape(n, 1024)
    logits = feat @ p["fc_w"].T + p["fc_b"]
    return jax.nn.softmax(logits, axis=1)


def kernel(x_nchw, init1_w, init1_b, init2_w, init2_b, init3_w, init3_b, in_3a_head_w, in_3a_head_b, in_3a_b2b_w, in_3a_b2b_b, in_3a_b3b_w, in_3a_b3b_b, in_3a_b4_w, in_3a_b4_b, in_3b_head_w, in_3b_head_b, in_3b_b2b_w, in_3b_b2b_b, in_3b_b3b_w, in_3b_b3b_b, in_3b_b4_w, in_3b_b4_b, in_4a_head_w, in_4a_head_b, in_4a_b2b_w, in_4a_b2b_b, in_4a_b3b_w, in_4a_b3b_b, in_4a_b4_w, in_4a_b4_b, in_4b_head_w, in_4b_head_b, in_4b_b2b_w, in_4b_b2b_b, in_4b_b3b_w, in_4b_b3b_b, in_4b_b4_w, in_4b_b4_b, in_4c_head_w, in_4c_head_b, in_4c_b2b_w, in_4c_b2b_b, in_4c_b3b_w, in_4c_b3b_b, in_4c_b4_w, in_4c_b4_b, in_4d_head_w, in_4d_head_b, in_4d_b2b_w, in_4d_b2b_b, in_4d_b3b_w, in_4d_b3b_b, in_4d_b4_w, in_4d_b4_b, in_4e_head_w, in_4e_head_b, in_4e_b2b_w, in_4e_b2b_b, in_4e_b3b_w, in_4e_b3b_b, in_4e_b4_w, in_4e_b4_b, in_5a_head_w, in_5a_head_b, in_5a_b2b_w, in_5a_b2b_b, in_5a_b3b_w, in_5a_b3b_b, in_5a_b4_w, in_5a_b4_b, in_5b_head_w, in_5b_head_b, in_5b_b2b_w, in_5b_b2b_b, in_5b_b3b_w, in_5b_b3b_b, in_5b_b4_w, in_5b_b4_b, fc_w, fc_b):
    p = {
        "init1_w": init1_w, "init1_b": init1_b,
        "init2_w": init2_w, "init2_b": init2_b,
        "init3_w": init3_w, "init3_b": init3_b,
        "3a": (in_3a_head_w, in_3a_head_b, in_3a_b2b_w, in_3a_b2b_b,
               in_3a_b3b_w, in_3a_b3b_b, in_3a_b4_w, in_3a_b4_b),
        "3b": (in_3b_head_w, in_3b_head_b, in_3b_b2b_w, in_3b_b2b_b,
               in_3b_b3b_w, in_3b_b3b_b, in_3b_b4_w, in_3b_b4_b),
        "4a": (in_4a_head_w, in_4a_head_b, in_4a_b2b_w, in_4a_b2b_b,
               in_4a_b3b_w, in_4a_b3b_b, in_4a_b4_w, in_4a_b4_b),
        "4b": (in_4b_head_w, in_4b_head_b, in_4b_b2b_w, in_4b_b2b_b,
               in_4b_b3b_w, in_4b_b3b_b, in_4b_b4_w, in_4b_b4_b),
        "4c": (in_4c_head_w, in_4c_head_b, in_4c_b2b_w, in_4c_b2b_b,
               in_4c_b3b_w, in_4c_b3b_b, in_4c_b4_w, in_4c_b4_b),
        "4d": (in_4d_head_w, in_4d_head_b, in_4d_b2b_w, in_4d_b2b_b,
               in_4d_b3b_w, in_4d_b3b_b, in_4d_b4_w, in_4d_b4_b),
        "4e": (in_4e_head_w, in_4e_head_b, in_4e_b2b_w, in_4e_b2b_b,
               in_4e_b3b_w, in_4e_b3b_b, in_4e_b4_w, in_4e_b4_b),
        "5a": (in_5a_head_w, in_5a_head_b, in_5a_b2b_w, in_5a_b2b_b,
               in_5a_b3b_w, in_5a_b3b_b, in_5a_b4_w, in_5a_b4_b),
        "5b": (in_5b_head_w, in_5b_head_b, in_5b_b2b_w, in_5b_b2b_b,
               in_5b_b3b_w, in_5b_b3b_b, in_5b_b4_w, in_5b_b4_b),
        "fc_w": fc_w, "fc_b": fc_b,
    }
    return _forward(x_nchw, p)
